# baseline (speedup 1.0000x reference)
import functools

import jax
import jax.numpy as jnp
from jax import lax
from jax.experimental import pallas as pl
from jax.experimental.pallas import tpu as pltpu
from jax.experimental.pallas import tpu_sc as plsc

N_USERS = 6000
N_ITEMS = 4000
N_NODES = N_USERS + N_ITEMS
N_EDGES = 320000
DIM = 128
NQ = DIM // 16

NC = 2
NS = 16
TILES = NC * NS
CH = 80
NCH = 128
NCH_A = NCH + 4
EPT = NCH * CH
E_ALLOC = TILES * NCH_A * CH
IDEPTH = 8
N_PAD = 10240
RPT = N_PAD // TILES
RPS = N_PAD // NS

_mesh = plsc.VectorSubcoreMesh(core_axis_name="c", subcore_axis_name="s")


@functools.partial(
    pl.kernel,
    out_type=jax.ShapeDtypeStruct((NC, N_PAD, DIM), jnp.float32),
    mesh=_mesh,
    scratch_types=[
        pltpu.VMEM((IDEPTH, CH), jnp.int32),
        pltpu.VMEM((IDEPTH, CH), jnp.int32),
        pltpu.VMEM((IDEPTH, CH), jnp.float32),
        pltpu.VMEM((CH, DIM), jnp.float32),
        pltpu.VMEM((CH, DIM), jnp.float32),
        pltpu.VMEM((CH, DIM), jnp.float32),
        pltpu.VMEM((CH, DIM), jnp.float32),
        pltpu.VMEM_SHARED((N_PAD, DIM), jnp.float32),
        pltpu.SemaphoreType.DMA,
        pltpu.SemaphoreType.DMA,
        pltpu.SemaphoreType.DMA,
        pltpu.SemaphoreType.DMA,
        pltpu.SemaphoreType.DMA,
    ],
)
def _spmm_layer(row_h, col_h, val_h, x_h, out_h,
                rowr, colr, valr, gb0, gb1, sb0, sb1, acc,
                sem_i, sem_g0, sem_g1, sem_s0, sem_s1):
    c = lax.axis_index("c")
    s = lax.axis_index("s")
    tid = c * NS + s
    gb = (gb0, gb1)
    sb = (sb0, sb1)
    sem_g = (sem_g0, sem_g1)
    sem_s = (sem_s0, sem_s1)

    def load_idx(j):
        slot = lax.rem(j, IDEPTH)
        pltpu.async_copy(row_h.at[tid, j], rowr.at[slot], sem_i)
        pltpu.async_copy(col_h.at[tid, j], colr.at[slot], sem_i)
        pltpu.async_copy(val_h.at[tid, j], valr.at[slot], sem_i)

    def drain_idx():
        pltpu.make_async_copy(row_h.at[tid, 0], rowr.at[0], sem_i).wait()
        pltpu.make_async_copy(col_h.at[tid, 0], colr.at[0], sem_i).wait()
        pltpu.make_async_copy(val_h.at[tid, 0], valr.at[0], sem_i).wait()

    def drain_gather(b):
        pltpu.make_async_copy(x_h.at[pl.ds(0, CH)], gb[b], sem_g[b]).wait()

    def drain_scatter(b):
        pltpu.make_async_copy(x_h.at[pl.ds(0, CH)], sb[b], sem_s[b]).wait()

    def zrow(r, _):
        for q in range(NQ):
            sb0[r, pl.ds(q * 16, 16)] = jnp.zeros((16,), jnp.float32)
        return _
    lax.fori_loop(0, CH, zrow, None)
    for bzero in range(RPS // CH):
        pltpu.sync_copy(sb0, acc.at[pl.ds(s * RPS + bzero * CH, CH)])

    for j in range(4):
        load_idx(j)
    drain_idx()
    drain_idx()
    plsc.subcore_barrier()
    pltpu.async_copy(x_h.at[colr.at[0]], gb0, sem_g0)
    pltpu.async_copy(x_h.at[colr.at[1]], gb1, sem_g1)

    def chunk_pair(cp, _):
        for b in range(2):
            j = cp * 2 + b
            slot = lax.rem(j, IDEPTH)
            drain_idx()
            load_idx(j + 4)
            drain_gather(b)

            @pl.when(cp >= 1)
            def _():
                drain_scatter(b)

            def group(g, _):
                vals = valr[slot, pl.ds(g * 16, 16)]
                for l in range(16):
                    v = vals[l]
                    e = g * 16 + l
                    for q in range(NQ):
                        sl = pl.ds(q * 16, 16)
                        sb[b][e, sl] = gb[b][e, sl] * v
                return _
            lax.fori_loop(0, CH // 16, group, None)

            pltpu.async_copy(sb[b], acc.at[rowr.at[slot]], sem_s[b],
                             add=True)
            slot2 = lax.rem(j + 2, IDEPTH)
            pltpu.async_copy(x_h.at[colr.at[slot2]], gb[b], sem_g[b])
        return _
    lax.fori_loop(0, NCH // 2, chunk_pair, None)

    drain_gather(0)
    drain_gather(1)
    drain_scatter(0)
    drain_scatter(1)

    plsc.subcore_barrier()
    pltpu.sync_copy(acc.at[pl.ds(s * RPS, RPS)],
                    out_h.at[c, pl.ds(s * RPS, RPS)])


_CB = 64


@functools.partial(
    pl.kernel,
    out_type=jax.ShapeDtypeStruct((N_PAD, DIM), jnp.float32),
    mesh=_mesh,
    scratch_types=[
        pltpu.VMEM((_CB, DIM), jnp.float32),
        pltpu.VMEM((_CB, DIM), jnp.float32),
    ],
)
def _combine(p_h, x_h, a, b):
    c = lax.axis_index("c")
    s = lax.axis_index("s")
    tid = c * NS + s
    for t in range(RPT // _CB):
        start = tid * RPT + t * _CB
        pltpu.sync_copy(p_h.at[0, pl.ds(start, _CB)], a)
        pltpu.sync_copy(p_h.at[1, pl.ds(start, _CB)], b)

        def rbody(r, _):
            for q in range(NQ):
                sl = pl.ds(q * 16, 16)
                a[r, sl] = a[r, sl] + b[r, sl]
            return _
        lax.fori_loop(0, _CB, rbody, None)
        pltpu.sync_copy(a, x_h.at[pl.ds(start, _CB)])


@functools.partial(
    pl.kernel,
    out_type=jax.ShapeDtypeStruct((N_PAD, DIM), jnp.float32),
    mesh=_mesh,
    scratch_types=[
        pltpu.VMEM((_CB, DIM), jnp.float32),
        pltpu.VMEM((_CB, DIM), jnp.float32),
        pltpu.VMEM((_CB, DIM), jnp.float32),
        pltpu.VMEM((_CB, DIM), jnp.float32),
        pltpu.VMEM((_CB, DIM), jnp.float32),
    ],
)
def _finalize(x0_h, x1_h, x2_h, p3_h, out_h, a, b, d, e, f):
    c = lax.axis_index("c")
    s = lax.axis_index("s")
    tid = c * NS + s
    for t in range(RPT // _CB):
        start = tid * RPT + t * _CB
        pltpu.sync_copy(x0_h.at[pl.ds(start, _CB)], a)
        pltpu.sync_copy(x1_h.at[pl.ds(start, _CB)], b)
        pltpu.sync_copy(x2_h.at[pl.ds(start, _CB)], d)
        pltpu.sync_copy(p3_h.at[0, pl.ds(start, _CB)], e)
        pltpu.sync_copy(p3_h.at[1, pl.ds(start, _CB)], f)

        def rbody(r, _):
            for q in range(NQ):
                sl = pl.ds(q * 16, 16)
                tot = (((a[r, sl] + b[r, sl]) + (d[r, sl] + e[r, sl]))
                       + f[r, sl])
                a[r, sl] = tot * 0.25
            return _
        lax.fori_loop(0, _CB, rbody, None)
        pltpu.sync_copy(a, out_h.at[pl.ds(start, _CB)])


def kernel(adj_indices, adj_values, user_weight, item_weight):
    row = adj_indices[0].astype(jnp.int32)
    col = adj_indices[1].astype(jnp.int32)
    val = adj_values.astype(jnp.float32)
    pad = TILES * EPT - N_EDGES

    def lay_out(arr, dt):
        full = jnp.concatenate([arr, jnp.zeros((pad,), dt)]) \
            .reshape(TILES, EPT)
        look = jnp.zeros((TILES, (NCH_A - NCH) * CH), dt)
        return jnp.concatenate([full, look], axis=1) \
            .reshape(TILES, NCH_A, CH)

    row_p = lay_out(row, jnp.int32)
    col_p = lay_out(col, jnp.int32)
    val_p = lay_out(val, jnp.float32)

    x0 = jnp.zeros((N_PAD, DIM), jnp.float32)
    x0 = x0.at[:N_USERS].set(user_weight)
    x0 = x0.at[N_USERS:N_NODES].set(item_weight)

    p1 = _spmm_layer(row_p, col_p, val_p, x0)
    x1 = _combine(p1)
    p2 = _spmm_layer(row_p, col_p, val_p, x1)
    x2 = _combine(p2)
    p3 = _spmm_layer(row_p, col_p, val_p, x2)
    fin = _finalize(x0, x1, x2, p3)

    return (fin[:N_USERS], fin[N_USERS:N_NODES])
